# baseline (speedup 1.0000x reference)
import functools

import jax
import jax.numpy as jnp
import numpy as np
from jax import lax
from jax.experimental import pallas as pl
from jax.experimental.pallas import tpu as pltpu
from jax.experimental.pallas import tpu_sc as plsc

N = 10000
C = 128
K = 32

NC, NS, L = 2, 16, 16
NW = NC * NS
NPW = 320
NPAD = NW * NPW
GN = 4
GR = GN * K
NG = NPW // GN
CH = C // L
KH = K // L

_PAD_IDX = np.asarray(
    (np.arange((NPAD - N) * K) % N).reshape(1, NPAD - N, K), np.int32)

_PERM = np.empty((C,), np.int32)
for _c in range(C // 32):
    for _j in range(16):
        _PERM[32 * _c + 2 * _j] = 32 * _c + _j
        _PERM[32 * _c + 2 * _j + 1] = 32 * _c + 16 + _j


def _tc_body(xT_ref, Wt_ref, aT_ref, whT_ref, st_ref):
    whT = jnp.dot(xT_ref[...], Wt_ref[...], preferred_element_type=jnp.float32)
    whT_ref[...] = whT
    st_ref[...] = jnp.dot(whT, aT_ref[...], preferred_element_type=jnp.float32)


_tc_call = pl.pallas_call(
    _tc_body,
    out_shape=[
        jax.ShapeDtypeStruct((NPAD, C), jnp.float32),
        jax.ShapeDtypeStruct((NPAD, 2), jnp.float32),
    ],
)


NB = 4


def _sc_body(whT_hbm, s_hbm, t_hbm, i0_hbm, i1_hbm, out_hbm,
             s_v, t_v, i0_v, i1_v, p_v, rows_v, o_v,
             sem0, sem1, sem2, sem3, osem0, osem1, osem2, osem3):
    sid = lax.axis_index("s")
    wid = sid * NC + lax.axis_index("c")

    pltpu.sync_copy(i0_hbm.at[wid], i0_v)
    sems = (sem0, sem1, sem2, sem3)
    osems = (osem0, osem1, osem2, osem3)

    def mk(g, slot):
        idx = i0_v.at[pl.ds(g * GR, GR)]
        return pltpu.make_async_copy(whT_hbm.at[idx], rows_v.at[slot], sems[slot])

    def mko(g, slot):
        return pltpu.make_async_copy(
            o_v.at[slot], out_hbm.at[pl.ds(wid * NPW + g * GN, GN)],
            osems[slot])

    for b in range(NB):
        mk(b, b).start()

    pltpu.sync_copy(s_hbm, s_v)
    pltpu.sync_copy(t_hbm, t_v)
    pltpu.sync_copy(i1_hbm.at[wid], i1_v)

    def e_body(n, carry):
        off = n * K
        es = []
        for kk in range(KH):
            ii0 = i0_v[pl.ds(off + kk * L, L)]
            ii1 = i1_v[pl.ds(off + kk * L, L)]
            e = plsc.load_gather(s_v, [ii1]) + plsc.load_gather(t_v, [ii0])
            es.append(jnp.where(e >= 0.0, e, 0.2 * e))
        m = jnp.max(es[0])
        for kk in range(1, KH):
            m = jnp.maximum(m, jnp.max(es[kk]))
        ps = [jnp.exp(e - m) for e in es]
        d = ps[0].sum()
        for kk in range(1, KH):
            d = d + ps[kk].sum()
        rv = 1.0 / jnp.full((L,), d)
        for kk in range(KH):
            p_v[pl.ds(off + kk * L, L)] = ps[kk] * rv
        return carry

    lax.fori_loop(0, NPW, e_body, 0)

    def g_body(gp, carry):
        for b in range(NB):
            g = gp * NB + b
            mk(g, b).wait()

            @pl.when(g >= NB)
            def _():
                mko(g - NB, b).wait()

            for j in range(GN):
                n = g * GN + j

                def k_body(k, acc):
                    w = plsc.load_gather(p_v, [jnp.full((L,), n * K + k, jnp.int32)])
                    return [acc[c] + w * rows_v[b, j * K + k, pl.ds(c * L, L)]
                            for c in range(CH)]

                acc = lax.fori_loop(
                    0, K, k_body, [jnp.zeros((L,), jnp.float32)] * CH,
                    unroll=8)
                for c in range(CH):
                    o_v[b, j, pl.ds(c * L, L)] = acc[c]

            mko(g, b).start()

            @pl.when(g + NB < NG)
            def _():
                mk(g + NB, b).start()

        return carry

    lax.fori_loop(0, NG // NB, g_body, 0)

    for b in range(NB):
        mko(NG - NB + b, b).wait()


_sc_call = functools.partial(
    pl.kernel,
    out_type=jax.ShapeDtypeStruct((NPAD, C), jnp.float32),
    mesh=plsc.VectorSubcoreMesh(
        core_axis_name="c", subcore_axis_name="s",
        num_cores=NC, num_subcores=NS),
    scratch_types=[
        pltpu.VMEM((NPAD,), jnp.float32),
        pltpu.VMEM((NPAD,), jnp.float32),
        pltpu.VMEM((NPW * K,), jnp.int32),
        pltpu.VMEM((NPW * K,), jnp.int32),
        pltpu.VMEM((NPW * K,), jnp.float32),
        pltpu.VMEM((4, GR, C), jnp.float32),
        pltpu.VMEM((4, GN, C), jnp.float32),
        pltpu.SemaphoreType.DMA,
        pltpu.SemaphoreType.DMA,
        pltpu.SemaphoreType.DMA,
        pltpu.SemaphoreType.DMA,
        pltpu.SemaphoreType.DMA,
        pltpu.SemaphoreType.DMA,
        pltpu.SemaphoreType.DMA,
        pltpu.SemaphoreType.DMA,
    ],
    compiler_params=pltpu.CompilerParams(needs_layout_passes=False),
)(_sc_body)


def kernel(x, edge_index, W, a):
    B = x.shape[0]
    xf = x.reshape(B * C, N)
    xT = jnp.pad(xf.T, ((0, NPAD - N), (0, 0)))
    Wt = W.T
    aT = a.reshape(2, C).T

    whT, st = _tc_call(xT, Wt, aT)
    s = st[:, 0]
    t = st[:, 1]

    idx = edge_index.astype(jnp.int32).reshape(2, N, K)
    pad_idx = jnp.broadcast_to(jnp.asarray(_PAD_IDX), (2, NPAD - N, K))
    idxp = jnp.concatenate([idx, pad_idx], axis=1)
    i0 = idxp[0].reshape(NW, NPW * K)
    i1 = idxp[1].reshape(NW, NPW * K)

    hT = _sc_call(whT, s, t, i0, i1)
    return hT[:N].T.reshape(B, C, N, 1)


# speedup vs baseline: 1.0412x; 1.0412x over previous
import functools

import jax
import jax.numpy as jnp
import numpy as np
from jax import lax
from jax.experimental import pallas as pl
from jax.experimental.pallas import tpu as pltpu
from jax.experimental.pallas import tpu_sc as plsc

N = 10000
C = 128
K = 32

NC, NS, L = 2, 16, 16
NW = NC * NS
NPW = 320
NPAD = NW * NPW
GN = 4
GR = GN * K
NG = NPW // GN
CH = C // L
KH = K // L

_PAD_IDX = np.asarray(
    (np.arange((NPAD - N) * K) % N).reshape(1, NPAD - N, K), np.int32)

_PERM = np.empty((C,), np.int32)
for _c in range(C // 32):
    for _j in range(16):
        _PERM[32 * _c + 2 * _j] = 32 * _c + _j
        _PERM[32 * _c + 2 * _j + 1] = 32 * _c + 16 + _j


def _tc_body(xT_ref, Wt_ref, aT_ref, whT_ref, st_ref):
    whT = jnp.dot(xT_ref[...], Wt_ref[...], preferred_element_type=jnp.float32)
    whT_ref[...] = whT
    st_ref[...] = jnp.dot(whT, aT_ref[...], preferred_element_type=jnp.float32)


_tc_call = pl.pallas_call(
    _tc_body,
    out_shape=[
        jax.ShapeDtypeStruct((NPAD, C), jnp.float32),
        jax.ShapeDtypeStruct((NPAD, 2), jnp.float32),
    ],
)


NB = 2
NPS = NPAD // NS


def _sc_body(whT_hbm, s_hbm, t_hbm, i0_hbm, i1_hbm, out_hbm,
             s_v, t_v, i0_v, i1_v, p_v, rows_v, o_v,
             sem0, sem1, osem0, osem1):
    sid = lax.axis_index("s")
    wid = sid * NC + lax.axis_index("c")

    pltpu.sync_copy(i0_hbm.at[wid], i0_v)
    sems = (sem0, sem1)
    osems = (osem0, osem1)

    def mk(g, slot):
        idx = i0_v.at[pl.ds(g * GR, GR)]
        return pltpu.make_async_copy(whT_hbm.at[idx], rows_v.at[slot], sems[slot])

    def mko(g, slot):
        return pltpu.make_async_copy(
            o_v.at[slot], out_hbm.at[pl.ds(wid * NPW + g * GN, GN)],
            osems[slot])

    for b in range(NB):
        mk(b, b).start()

    pltpu.sync_copy(s_hbm, s_v)
    pltpu.sync_copy(t_hbm, t_v)
    pltpu.sync_copy(i1_hbm.at[wid], i1_v)

    def e_body(n, carry):
        off = n * K
        es = []
        for kk in range(KH):
            ii0 = i0_v[pl.ds(off + kk * L, L)]
            ii1 = i1_v[pl.ds(off + kk * L, L)]
            e = plsc.load_gather(s_v, [ii1]) + plsc.load_gather(t_v, [ii0])
            es.append(jnp.where(e >= 0.0, e, 0.2 * e))
        m = jnp.max(es[0])
        for kk in range(1, KH):
            m = jnp.maximum(m, jnp.max(es[kk]))
        ps = [jnp.exp(e - m) for e in es]
        d = ps[0].sum()
        for kk in range(1, KH):
            d = d + ps[kk].sum()
        rv = 1.0 / jnp.full((L,), d)
        for kk in range(KH):
            p_v[pl.ds(off + kk * L, L)] = ps[kk] * rv
        return carry

    lax.fori_loop(0, NPW, e_body, 0)

    def g_body(gp, carry):
        for b in range(NB):
            g = gp * NB + b
            mk(g, b).wait()

            @pl.when(g >= NB)
            def _():
                mko(g - NB, b).wait()

            for j in range(GN):
                n = g * GN + j

                def k_body(k, acc):
                    w = plsc.load_gather(p_v, [jnp.full((L,), n * K + k, jnp.int32)])
                    return [acc[c] + w * rows_v[b, j * K + k, pl.ds(c * L, L)]
                            for c in range(CH)]

                acc = lax.fori_loop(
                    0, K, k_body, [jnp.zeros((L,), jnp.float32)] * CH)
                for c in range(CH):
                    o_v[b, j, pl.ds(c * L, L)] = acc[c]

            mko(g, b).start()

            @pl.when(g + NB < NG)
            def _():
                mk(g + NB, b).start()

        return carry

    lax.fori_loop(0, NG // NB, g_body, 0)

    for b in range(NB):
        mko(NG - NB + b, b).wait()


_sc_call = functools.partial(
    pl.kernel,
    out_type=jax.ShapeDtypeStruct((NPAD, C), jnp.float32),
    mesh=plsc.VectorSubcoreMesh(
        core_axis_name="c", subcore_axis_name="s",
        num_cores=NC, num_subcores=NS),
    scratch_types=[
        pltpu.VMEM((NPAD,), jnp.float32),
        pltpu.VMEM((NPAD,), jnp.float32),
        pltpu.VMEM((NPW * K,), jnp.int32),
        pltpu.VMEM((NPW * K,), jnp.int32),
        pltpu.VMEM((NPW * K,), jnp.float32),
        pltpu.VMEM((NB, GR, C), jnp.float32),
        pltpu.VMEM((NB, GN, C), jnp.float32),
        pltpu.SemaphoreType.DMA,
        pltpu.SemaphoreType.DMA,
        pltpu.SemaphoreType.DMA,
        pltpu.SemaphoreType.DMA,
    ],
    compiler_params=pltpu.CompilerParams(needs_layout_passes=False),
)(_sc_body)


def kernel(x, edge_index, W, a):
    B = x.shape[0]
    xf = x.reshape(B * C, N)
    xT = jnp.pad(xf.T, ((0, NPAD - N), (0, 0)))
    Wt = W.T
    aT = a.reshape(2, C).T

    whT, st = _tc_call(xT, Wt, aT)
    s = st[:, 0]
    t = st[:, 1]

    idx = edge_index.astype(jnp.int32).reshape(2, N, K)
    pad_idx = jnp.broadcast_to(jnp.asarray(_PAD_IDX), (2, NPAD - N, K))
    idxp = jnp.concatenate([idx, pad_idx], axis=1)
    i0 = idxp[0].reshape(NW, NPW * K)
    i1 = idxp[1].reshape(NW, NPW * K)

    hT = _sc_call(whT, s, t, i0, i1)
    return hT[:N].T.reshape(B, C, N, 1)
